# baseline (speedup 1.0000x reference)
import jax
import jax.numpy as jnp
from jax.experimental import pallas as pl
from jax.experimental.pallas import tpu as pltpu

NUM_CLASSES_K = 1000
BATCH_K = 4096
COLS_K = 20
SUB_K = 200
NSUB_K = NUM_CLASSES_K // SUB_K


def _onehot_body(xt_ref, o_hbm, vbuf, sems):
    t = pl.program_id(0)
    xv = xt_ref[pl.ds(t, 1), :]
    base_iota = jax.lax.broadcasted_iota(jnp.int32, (1, SUB_K, BATCH_K), 1)
    for j in range(NSUB_K):
        n = t * NSUB_K + j
        slot = jax.lax.rem(n, 4)

        @pl.when(n >= 4)
        def _wait():
            pn = n - 4
            pltpu.make_async_copy(
                vbuf.at[slot],
                o_hbm.at[pl.ds(pn // NSUB_K, 1), pl.ds((pn % NSUB_K) * SUB_K, SUB_K), :],
                sems.at[slot],
            ).wait()

        vbuf[slot] = jnp.where(
            xv[None] == base_iota + j * SUB_K, jnp.float32(1.0), jnp.float32(0.0)
        )
        pltpu.make_async_copy(
            vbuf.at[slot],
            o_hbm.at[pl.ds(t, 1), pl.ds(j * SUB_K, SUB_K), :],
            sems.at[slot],
        ).start()

    @pl.when(t == COLS_K - 1)
    def _drain():
        for k in range(4):
            pn = COLS_K * NSUB_K - 4 + k
            pltpu.make_async_copy(
                vbuf.at[jax.lax.rem(jnp.int32(pn), 4)],
                o_hbm.at[pl.ds(pn // NSUB_K, 1), pl.ds((pn % NSUB_K) * SUB_K, SUB_K), :],
                sems.at[jax.lax.rem(jnp.int32(pn), 4)],
            ).wait()


def kernel(x):
    xt = x.astype(jnp.int32).T
    out = pl.pallas_call(
        _onehot_body,
        grid=(COLS_K,),
        in_specs=[pl.BlockSpec((COLS_K, BATCH_K), lambda t: (0, 0))],
        out_specs=pl.BlockSpec(memory_space=pl.ANY),
        out_shape=jax.ShapeDtypeStruct((COLS_K, NUM_CLASSES_K, BATCH_K), jnp.float32),
        scratch_shapes=[
            pltpu.VMEM((4, 1, SUB_K, BATCH_K), jnp.float32),
            pltpu.SemaphoreType.DMA((4,)),
        ],
    )(xt)
    return out.transpose(2, 0, 1)
